# baseline (speedup 1.0000x reference)
import functools

import jax
import jax.numpy as jnp
from jax import lax
from jax.experimental import pallas as pl
from jax.experimental.pallas import tpu as pltpu
from jax.experimental.pallas import tpu_sc as plsc


@functools.lru_cache(maxsize=None)
def _build(B, NEG, V, D):
    info = plsc.get_sparse_core_info()
    NC, NS = info.num_cores, info.num_subcores
    NW = NC * NS
    K = 2 + NEG
    CH = 16
    ROWS = CH * K
    BW = B // NW
    n_chunks = BW // CH
    assert B % NW == 0 and BW % (2 * CH) == 0

    mesh = plsc.VectorSubcoreMesh(core_axis_name="c", subcore_axis_name="s")

    @functools.partial(
        pl.kernel,
        mesh=mesh,
        compiler_params=pltpu.CompilerParams(use_tc_tiling_on_sc=False),
        out_type=jax.ShapeDtypeStruct((B * K, D), jnp.float32),
        scratch_types=[
            pltpu.VMEM((ROWS,), jnp.int32),
            pltpu.VMEM((ROWS,), jnp.int32),
            pltpu.VMEM((ROWS, D), jnp.float32),
            pltpu.VMEM((ROWS, D), jnp.float32),
            pltpu.SemaphoreType.DMA,
            pltpu.SemaphoreType.DMA,
            pltpu.SemaphoreType.DMA,
            pltpu.SemaphoreType.DMA,
        ],
    )
    def skipgram(comb_hbm, wt_hbm, wc_hbm, out_hbm,
                 idx0, idx1, buf0, buf1, gsem0, gsem1, wsem0, wsem1):
        wid = lax.axis_index("s") * NC + lax.axis_index("c")
        wrow0 = wid * (BW * K)

        def do_chunk(g, c, idx_v, buf, gsem, wsem):
            row0 = wrow0 + c * ROWS

            @pl.when(g >= 1)
            def _():
                pltpu.make_async_copy(
                    buf, out_hbm.at[pl.ds(row0, ROWS)], wsem).wait()

            pltpu.sync_copy(comb_hbm.at[pl.ds(row0, ROWS)], idx_v)

            for v in range(ROWS // 16):
                vec = idx_v[pl.ds(v * 16, 16)]
                for j in range(16):
                    k = v * 16 + j
                    src = wt_hbm if k % K == 0 else wc_hbm
                    pltpu.make_async_copy(
                        src.at[pl.ds(vec[j], 1)],
                        buf.at[pl.ds(k, 1)], gsem).start()

            pltpu.make_async_copy(wt_hbm.at[pl.ds(0, ROWS)], buf, gsem).wait()
            pltpu.make_async_copy(
                buf, out_hbm.at[pl.ds(row0, ROWS)], wsem).start()

        def loop_body(g, carry):
            do_chunk(g, 2 * g, idx0, buf0, gsem0, wsem0)
            do_chunk(g, 2 * g + 1, idx1, buf1, gsem1, wsem1)
            return carry

        lax.fori_loop(0, n_chunks // 2, loop_body, 0)

        pltpu.make_async_copy(
            buf0, out_hbm.at[pl.ds(wrow0, ROWS)], wsem0).wait()
        pltpu.make_async_copy(
            buf1, out_hbm.at[pl.ds(wrow0, ROWS)], wsem1).wait()

    return skipgram


def kernel(target_words, context_words, negative_examples, W_target, W_context):
    B = target_words.shape[0]
    NEG = negative_examples.shape[1]
    V, D = W_target.shape
    tw = target_words.astype(jnp.int32)
    cw = context_words.astype(jnp.int32)
    ne = negative_examples.astype(jnp.int32)
    K = 2 + NEG
    comb = jnp.concatenate([tw[:, None], cw[:, None], ne], axis=1).reshape(B * K)
    fn = _build(B, NEG, V, D)
    out = fn(comb, W_target, W_context)
    return out.reshape(B, K, D)
